# baseline (speedup 1.0000x reference)
import functools

import jax
import jax.numpy as jnp
from jax import lax
from jax.experimental import pallas as pl
from jax.experimental.pallas import tpu as pltpu
from jax.experimental.pallas import tpu_sc as plsc

D = 64
L = 16
CHUNK = 128
NC, NS = 2, 16
NW = NC * NS

EPS = 1e-12


def _rsqrt_nr(x):
    i = plsc.bitcast(x, jnp.int32)
    y = plsc.bitcast(jnp.int32(0x5F3759DF) - (i >> 1), jnp.float32)
    y = y * (1.5 - 0.5 * x * y * y)
    y = y * (1.5 - 0.5 * x * y * y)
    y = y * (1.5 - 0.5 * x * y * y)
    return y


def _ln_rows(rows, w_v, b_v, n_rows):
    iota = lax.iota(jnp.int32, L)

    def group_body(gg, _):
        rvec = gg * L + iota
        s1 = jnp.zeros((L,), jnp.float32)
        s2 = jnp.zeros((L,), jnp.float32)
        for c in range(D):
            cvec = jnp.full((L,), c, jnp.int32)
            x = plsc.load_gather(rows, [rvec, cvec])
            s1 = s1 + x
            s2 = s2 + x * x
        mean = s1 * (1.0 / D)
        var = s2 * (1.0 / D) - mean * mean
        inv = _rsqrt_nr(var + EPS)
        for c in range(D):
            cvec = jnp.full((L,), c, jnp.int32)
            x = plsc.load_gather(rows, [rvec, cvec])
            y = (x - mean) * (inv * w_v[c]) + b_v[c]
            plsc.store_scatter(rows, [rvec, cvec], y)
        return 0

    lax.fori_loop(0, n_rows // L, group_body, 0)


def _make_kernel(n_blocks_total):
    blocks_per_w = n_blocks_total // NW

    mesh = plsc.VectorSubcoreMesh(core_axis_name="c", subcore_axis_name="s")

    @functools.partial(
        pl.kernel,
        out_type=jax.ShapeDtypeStruct((n_blocks_total * CHUNK, D), jnp.float32),
        mesh=mesh,
        compiler_params=pltpu.CompilerParams(
            needs_layout_passes=False, use_tc_tiling_on_sc=False
        ),
        scratch_types=[
            pltpu.VMEM((blocks_per_w, CHUNK), jnp.int32),
            pltpu.VMEM((CHUNK, D), jnp.float32),
            pltpu.SMEM((D,), jnp.float32),
            pltpu.SMEM((D,), jnp.float32),
            pltpu.VMEM((2, D), jnp.float32),
            pltpu.SemaphoreType.DMA,
        ],
    )
    def kern(idx_hbm, table_hbm, w_hbm, b_hbm, out_hbm, idx_v, rows_v, w_v, b_v, wb_stage, sem):
        wid = lax.axis_index("s") * NC + lax.axis_index("c")
        pltpu.sync_copy(idx_hbm.at[pl.ds(wid * blocks_per_w, blocks_per_w)], idx_v)
        pltpu.sync_copy(w_hbm, wb_stage.at[0])
        pltpu.sync_copy(b_hbm, wb_stage.at[1])
        for j in range(D // L):
            wv = wb_stage[0, pl.ds(L * j, L)]
            bv = wb_stage[1, pl.ds(L * j, L)]
            for i in range(L):
                w_v[L * j + i] = wv[i]
                b_v[L * j + i] = bv[i]

        def blk_body(g, _):
            pltpu.async_copy(table_hbm.at[idx_v.at[g]], rows_v, sem).wait()
            _ln_rows(rows_v, w_v, b_v, CHUNK)
            base = (wid * blocks_per_w + g) * CHUNK
            pltpu.sync_copy(rows_v, out_hbm.at[pl.ds(base, CHUNK)])
            return 0

        lax.fori_loop(0, blocks_per_w, blk_body, 0)

    return kern


def kernel(inputs, table, ln_weight, ln_bias):
    n_tokens, n_per = inputs.shape
    total = n_tokens * n_per
    n_blocks = total // CHUNK
    idx = inputs.reshape(n_blocks, CHUNK).astype(jnp.int32)
    out = _make_kernel(n_blocks)(idx, table, ln_weight, ln_bias)
    return out.reshape(n_tokens, n_per, D)


# speedup vs baseline: 3.0212x; 3.0212x over previous
import functools

import jax
import jax.numpy as jnp
from jax import lax
from jax.experimental import pallas as pl
from jax.experimental.pallas import tpu as pltpu
from jax.experimental.pallas import tpu_sc as plsc

D = 64
L = 16
CHUNK = 128
NC, NS = 2, 16
NW = NC * NS
NBUF = 4

EPS = 1e-12

_GATHER_DN = lax.GatherDimensionNumbers(
    offset_dims=(), collapsed_slice_dims=(0,), start_index_map=(0,)
)


def _lane_perm(x, perm2d):
    return lax.gather(
        x, perm2d, _GATHER_DN, (1,),
        mode=lax.GatherScatterMode.PROMISE_IN_BOUNDS,
    )


def _rsqrt_nr(x):
    i = plsc.bitcast(x, jnp.int32)
    y = plsc.bitcast(jnp.int32(0x5F3759DF) - (i >> 1), jnp.float32)
    xh = x * -0.5
    y = y * (xh * y * y + 1.5)
    y = y * (xh * y * y + 1.5)
    return y


def _make_kernel(n_blocks_total):
    bpw = n_blocks_total // NW

    mesh = plsc.VectorSubcoreMesh(core_axis_name="c", subcore_axis_name="s")

    @functools.partial(
        pl.kernel,
        out_type=jax.ShapeDtypeStruct((n_blocks_total * CHUNK, D), jnp.float32),
        mesh=mesh,
        compiler_params=pltpu.CompilerParams(
            needs_layout_passes=False, use_tc_tiling_on_sc=False
        ),
        scratch_types=[
            pltpu.VMEM((bpw, CHUNK), jnp.int32),
            pltpu.VMEM((NBUF, CHUNK, D), jnp.float32),
            pltpu.VMEM((2, D), jnp.float32),
            pltpu.SemaphoreType.DMA((NBUF,)),
            pltpu.SemaphoreType.DMA((NBUF,)),
        ],
    )
    def kern(idx_hbm, table_hbm, w_hbm, b_hbm, out_hbm, idx_v, rows_v, wb_v, gsem, osem):
        wid = lax.axis_index("s") * NC + lax.axis_index("c")
        pltpu.sync_copy(idx_hbm.at[pl.ds(wid * bpw, bpw)], idx_v)
        pltpu.sync_copy(w_hbm, wb_v.at[0])
        pltpu.sync_copy(b_hbm, wb_v.at[1])
        w_regs = [wb_v[0, pl.ds(L * j, L)] for j in range(D // L)]
        b_regs = [wb_v[1, pl.ds(L * j, L)] for j in range(D // L)]
        iota = lax.iota(jnp.int32, L)
        perms = [(iota ^ k)[:, None] for k in (1, 2, 4, 8)]

        pltpu.async_copy(table_hbm.at[idx_v.at[0]], rows_v.at[0], gsem.at[0])
        pltpu.async_copy(table_hbm.at[idx_v.at[1]], rows_v.at[1], gsem.at[1])

        def iter_body(g, _):
            b = jnp.bitwise_and(g, NBUF - 1)
            pltpu.make_async_copy(
                table_hbm.at[pl.ds(0, CHUNK)], rows_v.at[b], gsem.at[b]
            ).wait()

            @plsc.parallel_loop(0, CHUNK, step=1, unroll=2)
            def row(r):
                v = [rows_v[b, r, pl.ds(L * j, L)] for j in range(D // L)]
                t = (v[0] + v[1]) + (v[2] + v[3])
                q = (v[0] * v[0] + v[1] * v[1]) + (v[2] * v[2] + v[3] * v[3])
                for p in perms:
                    t = t + _lane_perm(t, p)
                    q = q + _lane_perm(q, p)
                mean = t * (1.0 / D)
                var = q * (1.0 / D) - mean * mean
                inv = _rsqrt_nr(var + EPS)
                for j in range(D // L):
                    rows_v[b, r, pl.ds(L * j, L)] = (
                        (v[j] - mean) * inv * w_regs[j] + b_regs[j]
                    )

            base = (wid * bpw + g) * CHUNK
            pltpu.async_copy(rows_v.at[b], out_hbm.at[pl.ds(base, CHUNK)], osem.at[b])

            g2 = g + 2
            b2 = jnp.bitwise_and(g2, NBUF - 1)

            @pl.when(g2 < bpw)
            def _():
                @pl.when(g >= 2)
                def _():
                    pltpu.make_async_copy(
                        rows_v.at[b2], out_hbm.at[pl.ds(0, CHUNK)], osem.at[b2]
                    ).wait()

                pltpu.async_copy(
                    table_hbm.at[idx_v.at[g2]], rows_v.at[b2], gsem.at[b2]
                )

            return 0

        lax.fori_loop(0, bpw, iter_body, 0)
        for b in range(NBUF):
            pltpu.make_async_copy(
                rows_v.at[b], out_hbm.at[pl.ds(0, CHUNK)], osem.at[b]
            ).wait()

    return kern


def kernel(inputs, table, ln_weight, ln_bias):
    n_tokens, n_per = inputs.shape
    total = n_tokens * n_per
    n_blocks = total // CHUNK
    idx = inputs.reshape(n_blocks, CHUNK).astype(jnp.int32)
    out = _make_kernel(n_blocks)(idx, table, ln_weight, ln_bias)
    return out.reshape(n_tokens, n_per, D)


# speedup vs baseline: 4.2656x; 1.4119x over previous
import functools

import jax
import jax.numpy as jnp
from jax import lax
from jax.experimental import pallas as pl
from jax.experimental.pallas import tpu as pltpu
from jax.experimental.pallas import tpu_sc as plsc

D = 64
L = 16
CHUNK = 128
PADW = 129
NC, NS = 2, 16
NW = NC * NS
NBUF = 4

EPS = 1e-12

_GATHER_DN = lax.GatherDimensionNumbers(
    offset_dims=(), collapsed_slice_dims=(0,), start_index_map=(0,)
)


def _lane_perm(x, perm2d):
    return lax.gather(
        x, perm2d, _GATHER_DN, (1,),
        mode=lax.GatherScatterMode.PROMISE_IN_BOUNDS,
    )


def _rsqrt_nr(x):
    i = plsc.bitcast(x, jnp.int32)
    y = plsc.bitcast(jnp.int32(0x5F3759DF) - (i >> 1), jnp.float32)
    xh = x * -0.5
    y = y * (xh * y * y + 1.5)
    y = y * (xh * y * y + 1.5)
    return y


def _make_kernel(n_tokens, n_slots):
    n_iblk = n_tokens // CHUNK
    n_chunks = n_slots * n_iblk
    bpw = n_chunks // NW

    mesh = plsc.VectorSubcoreMesh(core_axis_name="c", subcore_axis_name="s")

    @functools.partial(
        pl.kernel,
        out_type=jax.ShapeDtypeStruct(
            (n_slots, D // 8, n_iblk, 8, CHUNK), jnp.float32
        ),
        mesh=mesh,
        compiler_params=pltpu.CompilerParams(
            needs_layout_passes=False, use_tc_tiling_on_sc=False
        ),
        scratch_types=[
            pltpu.VMEM((bpw, CHUNK), jnp.int32),
            pltpu.VMEM((NBUF, CHUNK, D), jnp.float32),
            pltpu.VMEM((NBUF, D, PADW), jnp.float32),
            pltpu.VMEM((D,), jnp.float32),
            pltpu.VMEM((D,), jnp.float32),
            pltpu.SemaphoreType.DMA((NBUF,)),
            pltpu.SemaphoreType.DMA((NBUF,)),
        ],
    )
    def kern(idx_hbm, table_hbm, w_hbm, b_hbm, out_hbm,
             idx_v, rows_v, tbuf_v, w_v, b_v, gsem, osem):
        wid = lax.axis_index("s") * NC + lax.axis_index("c")
        pltpu.sync_copy(idx_hbm.at[wid], idx_v)
        pltpu.sync_copy(w_hbm, w_v)
        pltpu.sync_copy(b_hbm, b_v)
        w_regs = [w_v[pl.ds(L * j, L)] for j in range(D // L)]
        b_regs = [b_v[pl.ds(L * j, L)] for j in range(D // L)]
        iota = lax.iota(jnp.int32, L)
        perms = [(iota ^ k)[:, None] for k in (1, 2, 4, 8)]
        cvecs = [L * j + iota for j in range(D // L)]

        pltpu.async_copy(table_hbm.at[idx_v.at[0]], rows_v.at[0], gsem.at[0])
        pltpu.async_copy(table_hbm.at[idx_v.at[1]], rows_v.at[1], gsem.at[1])

        def iter_body(g, _):
            b = jnp.bitwise_and(g, NBUF - 1)
            pltpu.make_async_copy(
                table_hbm.at[pl.ds(0, CHUNK)], rows_v.at[b], gsem.at[b]
            ).wait()

            @plsc.parallel_loop(0, CHUNK, step=1, unroll=2)
            def row(r):
                v = [rows_v[b, r, pl.ds(L * j, L)] for j in range(D // L)]
                t = (v[0] + v[1]) + (v[2] + v[3])
                q = (v[0] * v[0] + v[1] * v[1]) + (v[2] * v[2] + v[3] * v[3])
                for p in perms:
                    t = t + _lane_perm(t, p)
                    q = q + _lane_perm(q, p)
                mean = t * (1.0 / D)
                var = q * (1.0 / D) - mean * mean
                inv = _rsqrt_nr(var + EPS)
                rvec = jnp.full((L,), r, jnp.int32)
                for j in range(D // L):
                    y = (v[j] - mean) * inv * w_regs[j] + b_regs[j]
                    plsc.store_scatter(tbuf_v.at[b], [cvecs[j], rvec], y)

            kglob = wid * bpw + g
            j_slot = kglob // n_iblk
            iblk = lax.rem(kglob, n_iblk)
            for ch in range(D // 8):
                pltpu.async_copy(
                    tbuf_v.at[b, pl.ds(8 * ch, 8), pl.ds(0, CHUNK)],
                    out_hbm.at[j_slot, ch, iblk],
                    osem.at[b],
                )

            g2 = g + 2
            b2 = jnp.bitwise_and(g2, NBUF - 1)

            @pl.when(g2 < bpw)
            def _():
                @pl.when(g >= 2)
                def _():
                    for ch in range(D // 8):
                        pltpu.make_async_copy(
                            tbuf_v.at[b2, pl.ds(0, 8), pl.ds(0, CHUNK)],
                            out_hbm.at[0, 0, 0],
                            osem.at[b2],
                        ).wait()

                pltpu.async_copy(
                    table_hbm.at[idx_v.at[g2]], rows_v.at[b2], gsem.at[b2]
                )

            return 0

        lax.fori_loop(0, bpw, iter_body, 0)
        for b in range(NBUF):
            for ch in range(D // 8):
                pltpu.make_async_copy(
                    tbuf_v.at[b, pl.ds(0, 8), pl.ds(0, CHUNK)],
                    out_hbm.at[0, 0, 0],
                    osem.at[b],
                ).wait()

    return kern


def kernel(inputs, table, ln_weight, ln_bias):
    n_tokens, n_slots = inputs.shape
    idx = inputs.T.reshape(NW, -1, CHUNK).astype(jnp.int32)
    out5 = _make_kernel(n_tokens, n_slots)(idx, table, ln_weight, ln_bias)
    return out5.transpose(2, 4, 0, 1, 3).reshape(n_tokens, n_slots, D)


# speedup vs baseline: 4.5562x; 1.0681x over previous
import functools

import jax
import jax.numpy as jnp
from jax import lax
from jax.experimental import pallas as pl
from jax.experimental.pallas import tpu as pltpu
from jax.experimental.pallas import tpu_sc as plsc

D = 64
DP = 128
L = 16
CHUNK = 128
PADW = 129
NC, NS = 2, 16
NW = NC * NS
NBUF = 4

EPS = 1e-12

_GATHER_DN = lax.GatherDimensionNumbers(
    offset_dims=(), collapsed_slice_dims=(0,), start_index_map=(0,)
)


def _lane_perm(x, perm2d):
    return lax.gather(
        x, perm2d, _GATHER_DN, (1,),
        mode=lax.GatherScatterMode.PROMISE_IN_BOUNDS,
    )


def _rsqrt_nr(x):
    i = plsc.bitcast(x, jnp.int32)
    y = plsc.bitcast(jnp.int32(0x5F3759DF) - (i >> 1), jnp.float32)
    xh = x * -0.5
    y = y * (xh * y * y + 1.5)
    y = y * (xh * y * y + 1.5)
    return y


def _make_kernel(n_tokens, n_slots):
    n_iblk = n_tokens // CHUNK
    n_chunks = n_slots * n_iblk
    bpw = n_chunks // NW

    mesh = plsc.VectorSubcoreMesh(core_axis_name="c", subcore_axis_name="s")

    @functools.partial(
        pl.kernel,
        out_type=jax.ShapeDtypeStruct(
            (n_slots, D // 8, n_iblk, 8, CHUNK), jnp.float32
        ),
        mesh=mesh,
        compiler_params=pltpu.CompilerParams(
            needs_layout_passes=False, use_tc_tiling_on_sc=False
        ),
        scratch_types=[
            pltpu.VMEM((bpw, CHUNK), jnp.int32),
            pltpu.VMEM((NBUF, CHUNK, DP), jnp.float32),
            pltpu.VMEM((NBUF, D, PADW), jnp.float32),
            pltpu.VMEM((D,), jnp.float32),
            pltpu.VMEM((D,), jnp.float32),
            pltpu.SemaphoreType.DMA((NBUF,)),
            pltpu.SemaphoreType.DMA((NBUF,)),
        ],
    )
    def kern(idx_hbm, table_hbm, w_hbm, b_hbm, out_hbm,
             idx_v, rows_v, tbuf_v, w_v, b_v, gsem, osem):
        wid = lax.axis_index("s") * NC + lax.axis_index("c")
        pltpu.sync_copy(idx_hbm.at[wid], idx_v)
        pltpu.sync_copy(w_hbm, w_v)
        pltpu.sync_copy(b_hbm, b_v)
        w_regs = [w_v[pl.ds(L * j, L)] for j in range(D // L)]
        b_regs = [b_v[pl.ds(L * j, L)] for j in range(D // L)]
        iota = lax.iota(jnp.int32, L)
        perms = [(iota ^ k)[:, None] for k in (1, 2, 4, 8)]
        cvecs = [L * j + iota for j in range(D // L)]

        pltpu.async_copy(table_hbm.at[idx_v.at[0]], rows_v.at[0], gsem.at[0])
        pltpu.async_copy(table_hbm.at[idx_v.at[1]], rows_v.at[1], gsem.at[1])

        def iter_body(g, _):
            b = jnp.bitwise_and(g, NBUF - 1)
            pltpu.make_async_copy(
                table_hbm.at[pl.ds(0, CHUNK)], rows_v.at[b], gsem.at[b]
            ).wait()

            @plsc.parallel_loop(0, CHUNK, step=1, unroll=2)
            def row(r):
                v = [rows_v[b, r, pl.ds(L * j, L)] for j in range(D // L)]
                t = (v[0] + v[1]) + (v[2] + v[3])
                q = (v[0] * v[0] + v[1] * v[1]) + (v[2] * v[2] + v[3] * v[3])
                for p in perms:
                    t = t + _lane_perm(t, p)
                    q = q + _lane_perm(q, p)
                mean = t * (1.0 / D)
                var = q * (1.0 / D) - mean * mean
                inv = _rsqrt_nr(var + EPS)
                rvec = jnp.full((L,), r, jnp.int32)
                for j in range(D // L):
                    y = (v[j] - mean) * inv * w_regs[j] + b_regs[j]
                    plsc.store_scatter(tbuf_v.at[b], [cvecs[j], rvec], y)

            kglob = wid * bpw + g
            j_slot = kglob // n_iblk
            iblk = lax.rem(kglob, n_iblk)
            for ch in range(D // 8):
                pltpu.async_copy(
                    tbuf_v.at[b, pl.ds(8 * ch, 8), pl.ds(0, CHUNK)],
                    out_hbm.at[j_slot, ch, iblk],
                    osem.at[b],
                )

            g2 = g + 2
            b2 = jnp.bitwise_and(g2, NBUF - 1)

            @pl.when(g2 < bpw)
            def _():
                @pl.when(g >= 2)
                def _():
                    for ch in range(D // 8):
                        pltpu.make_async_copy(
                            tbuf_v.at[b2, pl.ds(0, 8), pl.ds(0, CHUNK)],
                            out_hbm.at[0, 0, 0],
                            osem.at[b2],
                        ).wait()

                pltpu.async_copy(
                    table_hbm.at[idx_v.at[g2]], rows_v.at[b2], gsem.at[b2]
                )

            return 0

        lax.fori_loop(0, bpw, iter_body, 0)
        for b in range(NBUF):
            for ch in range(D // 8):
                pltpu.make_async_copy(
                    tbuf_v.at[b, pl.ds(0, 8), pl.ds(0, CHUNK)],
                    out_hbm.at[0, 0, 0],
                    osem.at[b],
                ).wait()

    return kern


def kernel(inputs, table, ln_weight, ln_bias):
    n_tokens, n_slots = inputs.shape
    idx = inputs.T.reshape(NW, -1, CHUNK).astype(jnp.int32)
    table_p = jnp.pad(table, ((0, 0), (0, DP - D)))
    out5 = _make_kernel(n_tokens, n_slots)(idx, table_p, ln_weight, ln_bias)
    return out5.transpose(2, 4, 0, 1, 3).reshape(n_tokens, n_slots, D)


# speedup vs baseline: 4.6725x; 1.0255x over previous
import functools

import jax
import jax.numpy as jnp
from jax import lax
from jax.experimental import pallas as pl
from jax.experimental.pallas import tpu as pltpu
from jax.experimental.pallas import tpu_sc as plsc

D = 64
DP = 128
L = 16
CHUNK = 128
PADW = 129
NC, NS = 2, 16
NW = NC * NS
NBUF = 4

EPS = 1e-12

_GATHER_DN = lax.GatherDimensionNumbers(
    offset_dims=(), collapsed_slice_dims=(0,), start_index_map=(0,)
)


def _lane_perm(x, perm2d):
    return lax.gather(
        x, perm2d, _GATHER_DN, (1,),
        mode=lax.GatherScatterMode.PROMISE_IN_BOUNDS,
    )


def _rsqrt_nr(x):
    i = plsc.bitcast(x, jnp.int32)
    y = plsc.bitcast(jnp.int32(0x5F3759DF) - (i >> 1), jnp.float32)
    xh = x * -0.5
    y = y * (xh * y * y + 1.5)
    return y


def _make_kernel(n_tokens, n_slots):
    n_iblk = n_tokens // CHUNK
    n_chunks = n_slots * n_iblk
    bpw = n_chunks // NW

    mesh = plsc.VectorSubcoreMesh(core_axis_name="c", subcore_axis_name="s")

    @functools.partial(
        pl.kernel,
        out_type=jax.ShapeDtypeStruct(
            (n_slots, D // 8, n_iblk, 8, CHUNK), jnp.float32
        ),
        mesh=mesh,
        compiler_params=pltpu.CompilerParams(
            needs_layout_passes=False, use_tc_tiling_on_sc=False
        ),
        scratch_types=[
            pltpu.VMEM((bpw, CHUNK), jnp.int32),
            pltpu.VMEM((NBUF, CHUNK, DP), jnp.float32),
            pltpu.VMEM((NBUF, D, PADW), jnp.float32),
            pltpu.VMEM((D,), jnp.float32),
            pltpu.VMEM((D,), jnp.float32),
            pltpu.SemaphoreType.DMA((NBUF,)),
            pltpu.SemaphoreType.DMA((NBUF,)),
        ],
    )
    def kern(idx_hbm, table_hbm, w_hbm, b_hbm, out_hbm,
             idx_v, rows_v, tbuf_v, w_v, b_v, gsem, osem):
        wid = lax.axis_index("s") * NC + lax.axis_index("c")
        pltpu.sync_copy(idx_hbm.at[wid], idx_v)
        pltpu.sync_copy(w_hbm, w_v)
        pltpu.sync_copy(b_hbm, b_v)
        w_regs = [w_v[pl.ds(L * j, L)] for j in range(D // L)]
        b_regs = [b_v[pl.ds(L * j, L)] for j in range(D // L)]
        iota = lax.iota(jnp.int32, L)
        perms = [(iota ^ k)[:, None] for k in (1, 2, 4, 8)]
        cvecs = [L * j + iota for j in range(D // L)]

        pltpu.async_copy(table_hbm.at[idx_v.at[0]], rows_v.at[0], gsem.at[0])
        pltpu.async_copy(table_hbm.at[idx_v.at[1]], rows_v.at[1], gsem.at[1])

        def iter_body(g, _):
            b = jnp.bitwise_and(g, NBUF - 1)
            pltpu.make_async_copy(
                table_hbm.at[pl.ds(0, CHUNK)], rows_v.at[b], gsem.at[b]
            ).wait()

            @plsc.parallel_loop(0, CHUNK, step=1, unroll=4)
            def row(r):
                v = [rows_v[b, r, pl.ds(L * j, L)] for j in range(D // L)]
                t = (v[0] + v[1]) + (v[2] + v[3])
                q = (v[0] * v[0] + v[1] * v[1]) + (v[2] * v[2] + v[3] * v[3])
                for p in perms:
                    t = t + _lane_perm(t, p)
                    q = q + _lane_perm(q, p)
                mean = t * (1.0 / D)
                var = q * (1.0 / D) - mean * mean
                inv = _rsqrt_nr(var + EPS)
                rvec = jnp.full((L,), r, jnp.int32)
                for j in range(D // L):
                    y = (v[j] - mean) * inv * w_regs[j] + b_regs[j]
                    plsc.store_scatter(tbuf_v.at[b], [cvecs[j], rvec], y)

            kglob = wid * bpw + g
            j_slot = kglob // n_iblk
            iblk = lax.rem(kglob, n_iblk)
            for ch in range(D // 8):
                pltpu.async_copy(
                    tbuf_v.at[b, pl.ds(8 * ch, 8), pl.ds(0, CHUNK)],
                    out_hbm.at[j_slot, ch, iblk],
                    osem.at[b],
                )

            g2 = g + 2
            b2 = jnp.bitwise_and(g2, NBUF - 1)

            @pl.when(g2 < bpw)
            def _():
                @pl.when(g >= 2)
                def _():
                    for ch in range(D // 8):
                        pltpu.make_async_copy(
                            tbuf_v.at[b2, pl.ds(0, 8), pl.ds(0, CHUNK)],
                            out_hbm.at[0, 0, 0],
                            osem.at[b2],
                        ).wait()

                pltpu.async_copy(
                    table_hbm.at[idx_v.at[g2]], rows_v.at[b2], gsem.at[b2]
                )

            return 0

        lax.fori_loop(0, bpw, iter_body, 0)
        for b in range(NBUF):
            for ch in range(D // 8):
                pltpu.make_async_copy(
                    tbuf_v.at[b, pl.ds(0, 8), pl.ds(0, CHUNK)],
                    out_hbm.at[0, 0, 0],
                    osem.at[b],
                ).wait()

    return kern


def kernel(inputs, table, ln_weight, ln_bias):
    n_tokens, n_slots = inputs.shape
    idx = inputs.T.reshape(NW, -1, CHUNK).astype(jnp.int32)
    table_p = jnp.pad(table, ((0, 0), (0, DP - D)))
    out5 = _make_kernel(n_tokens, n_slots)(idx, table_p, ln_weight, ln_bias)
    return out5.transpose(2, 4, 0, 1, 3).reshape(n_tokens, n_slots, D)
